# baseline (speedup 1.0000x reference)
import functools

import jax
import jax.numpy as jnp
import numpy as np
from jax.experimental import pallas as pl
from jax.experimental.pallas import tpu as pltpu
from jax.experimental.pallas import tpu_sc as plsc

_ROWS = 64
_V = 100000
_BK = 2048
_NB = -(-_V // _BK)
_K_SC = 6
_NB_HEAD = _NB - _K_SC
_V_HEAD = _NB_HEAD * _BK
_V_SC = _K_SC * _BK

_K1 = np.uint32(0)
_K2 = np.uint32(42)
_K3 = np.uint32(int(_K1) ^ int(_K2) ^ 0x1BD11BDA)
_TINY = np.float32(np.finfo(np.float32).tiny)
_ROT_B = (17, 29, 16, 24)
_ROT_A = (13, 15, 26, 6)
_KS = (_K1, _K2, _K3)

_SC_IDX = (np.arange(_ROWS, dtype=np.uint32)[:, None] * np.uint32(_V)
           + np.arange(_V_SC, dtype=np.uint32)[None, :]
           + np.uint32(_V_HEAD + 42))


def _rotl(x, d):
    return (x << np.uint32(d)) | (x >> np.uint32(32 - d))


def _threefry_bits(x1):
    x0 = x1
    x1 = _rotl(x1, 13)
    x1 = x0 ^ x1
    rounds = ((15, 26, 6), _ROT_B, _ROT_A, _ROT_B, _ROT_A)
    for r in range(5):
        for d in rounds[r]:
            x0 = x0 + x1
            x1 = _rotl(x1, d)
            x1 = x0 ^ x1
        x0 = x0 + _KS[(r + 1) % 3]
        x1 = x1 + np.uint32((int(_KS[(r + 2) % 3]) + r + 1) & 0xFFFFFFFF)
    return x0 ^ x1


def _gumbel_from_bits(bits, logits):
    float_bits = (bits >> np.uint32(9)) | np.uint32(0x3F800000)
    f = jax.lax.bitcast_convert_type(float_bits, jnp.float32) - np.float32(1.0)
    u = jnp.maximum(f, _TINY)
    g = -jnp.log(-jnp.log(u))
    return g + logits


def _gumbel_plus(logits, x1_init):
    return _gumbel_from_bits(_threefry_bits(x1_init), logits)


_SC_MESH = plsc.VectorSubcoreMesh(core_axis_name="core",
                                  subcore_axis_name="subcore")
_SC_DMA_BLOCK = (8, 1536)


def _sc_bits(idx):
    @functools.partial(
        pl.kernel,
        out_type=jax.ShapeDtypeStruct((_ROWS, _V_SC), jnp.uint32),
        mesh=_SC_MESH,
    )
    def sc_kernel(i_hbm, o_hbm):
        def body(i_vmem, o_vmem):
            @pl.loop(0, _SC_DMA_BLOCK[0])
            def _(r):
                @pl.loop(0, _SC_DMA_BLOCK[1], step=64)
                def _(c):
                    for t in range(4):
                        slc = (pl.ds(r, 1), pl.ds(c + t * 16, 16))
                        o_vmem.at[slc][...] = _threefry_bits(
                            i_vmem.at[slc][...])

        pltpu.emit_pipeline(
            body,
            grid=(_ROWS // _SC_DMA_BLOCK[0], _V_SC // _SC_DMA_BLOCK[1]),
            in_specs=[pl.BlockSpec(_SC_DMA_BLOCK, lambda i, j: (i, j))],
            out_specs=[pl.BlockSpec(_SC_DMA_BLOCK, lambda i, j: (i, j))],
            core_axis_name=("core", "subcore"),
            dimension_semantics=(pltpu.PARALLEL, pltpu.PARALLEL),
        )(i_hbm, o_hbm)

    return sc_kernel(idx)


def _head_kernel(x_ref, val_ref, idx_ref, x1_0, acc_val, acc_blk,
                 *, bk, nb, v, rows):
    j = pl.program_id(0)

    @pl.when(j == 0)
    def _():
        cols = jax.lax.broadcasted_iota(jnp.int32, (rows, bk), 1)
        r_iota = jax.lax.broadcasted_iota(jnp.int32, (rows, bk), 0)
        base = (r_iota * v + cols).astype(jnp.uint32) + np.uint32(42)
        x1_0[...] = base
        acc_val[...] = _gumbel_plus(x_ref[...], base)
        acc_blk[...] = jnp.zeros((rows, bk), jnp.int32)

    @pl.when(j > 0)
    def _():
        val = _gumbel_plus(x_ref[...], x1_0[...] + (j * bk).astype(jnp.uint32))
        better = val > acc_val[...]
        acc_blk[...] = jnp.where(better, j, acc_blk[...])
        acc_val[...] = jnp.maximum(val, acc_val[...])

    @pl.when(j == nb - 1)
    def _():
        cols = jax.lax.broadcasted_iota(jnp.int32, (rows, bk), 1)
        m = jnp.max(acc_val[...], axis=1, keepdims=True)
        cand = jnp.where(acc_val[...] == m, acc_blk[...] * bk + cols,
                         jnp.int32(2**31 - 1))
        val_ref[...] = m
        idx_ref[...] = jnp.min(cand, axis=1, keepdims=True)


def _tc_head(logits):
    rows, bk, nb = _ROWS, _BK, _NB_HEAD
    return pl.pallas_call(
        functools.partial(_head_kernel, bk=bk, nb=nb, v=_V, rows=rows),
        grid=(nb,),
        in_specs=[pl.BlockSpec((rows, bk), lambda j: (0, j))],
        out_specs=[pl.BlockSpec((rows, 1), lambda j: (0, 0)),
                   pl.BlockSpec((rows, 1), lambda j: (0, 0))],
        out_shape=[jax.ShapeDtypeStruct((rows, 1), jnp.float32),
                   jax.ShapeDtypeStruct((rows, 1), jnp.int32)],
        scratch_shapes=[
            pltpu.VMEM((rows, bk), jnp.uint32),
            pltpu.VMEM((rows, bk), jnp.float32),
            pltpu.VMEM((rows, bk), jnp.int32),
        ],
    )(logits)


def _tail_kernel(x_ref, b_ref, hval_ref, hidx_ref, o_ref, acc_val, acc_blk,
                 *, bk, nb, v, v0, rows):
    j = pl.program_id(0)

    @pl.when(j == 0)
    def _():
        acc_val[...] = _gumbel_from_bits(b_ref[...], x_ref[...])
        acc_blk[...] = jnp.zeros((rows, bk), jnp.int32)

    @pl.when(jnp.logical_and(j > 0, j < nb - 1))
    def _():
        val = _gumbel_from_bits(b_ref[...], x_ref[...])
        better = val > acc_val[...]
        acc_blk[...] = jnp.where(better, j, acc_blk[...])
        acc_val[...] = jnp.maximum(val, acc_val[...])

    @pl.when(j == nb - 1)
    def _():
        cols = jax.lax.broadcasted_iota(jnp.int32, (rows, bk), 1)
        val = _gumbel_from_bits(b_ref[...], x_ref[...])
        val = jnp.where(v0 + j * bk + cols < v, val, -jnp.inf)
        better = val > acc_val[...]
        a_blk = jnp.where(better, j, acc_blk[...])
        a_val = jnp.maximum(val, acc_val[...])
        m = jnp.max(a_val, axis=1, keepdims=True)
        cand = jnp.where(a_val == m, v0 + a_blk * bk + cols,
                         jnp.int32(2**31 - 1))
        t_idx = jnp.min(cand, axis=1, keepdims=True)
        pick_head = hval_ref[...] >= m
        res = jnp.where(pick_head, hidx_ref[...], t_idx)
        o_ref[...] = res.T


def _tc_tail(logits, bits, head_val, head_idx):
    rows, bk, nb = _ROWS, _BK, _K_SC
    return pl.pallas_call(
        functools.partial(_tail_kernel, bk=bk, nb=nb, v=_V, v0=_V_HEAD,
                          rows=rows),
        grid=(nb,),
        in_specs=[pl.BlockSpec((rows, bk), lambda j: (0, j + _NB_HEAD)),
                  pl.BlockSpec((rows, bk), lambda j: (0, j)),
                  pl.BlockSpec((rows, 1), lambda j: (0, 0)),
                  pl.BlockSpec((rows, 1), lambda j: (0, 0))],
        out_specs=pl.BlockSpec((1, rows), lambda j: (0, 0)),
        out_shape=jax.ShapeDtypeStruct((1, rows), jnp.int32),
        scratch_shapes=[
            pltpu.VMEM((rows, bk), jnp.float32),
            pltpu.VMEM((rows, bk), jnp.int32),
        ],
    )(logits, bits, head_val, head_idx)


def kernel(logits):
    rows, v = logits.shape
    bits = _sc_bits(jnp.asarray(_SC_IDX))
    head_val, head_idx = _tc_head(logits)
    out = _tc_tail(logits, bits, head_val, head_idx)
    return out.reshape(rows)


# speedup vs baseline: 2.3662x; 2.3662x over previous
import functools

import jax
import jax.numpy as jnp
import numpy as np
from jax.experimental import pallas as pl
from jax.experimental.pallas import tpu as pltpu
from jax.experimental.pallas import tpu_sc as plsc

_ROWS = 64
_V = 100000
_BK = 2048
_NB = -(-_V // _BK)
_K_SC = 5
_NB_HEAD = _NB - _K_SC
_V_HEAD = _NB_HEAD * _BK
_V_SC = _K_SC * _BK

_K1 = np.uint32(0)
_K2 = np.uint32(42)
_K3 = np.uint32(int(_K1) ^ int(_K2) ^ 0x1BD11BDA)
_TINY = np.float32(np.finfo(np.float32).tiny)
_ROT_B = (17, 29, 16, 24)
_ROT_A = (13, 15, 26, 6)
_KS = (_K1, _K2, _K3)

_SC_IDX = (np.arange(_ROWS, dtype=np.uint32)[:, None] * np.uint32(_V)
           + np.arange(_V_SC, dtype=np.uint32)[None, :]
           + np.uint32(_V_HEAD + 42))


def _rotl(x, d):
    return (x << np.uint32(d)) | (x >> np.uint32(32 - d))


def _threefry_bits(x1):
    x0 = x1
    x1 = _rotl(x1, 13)
    x1 = x0 ^ x1
    rounds = ((15, 26, 6), _ROT_B, _ROT_A, _ROT_B, _ROT_A)
    for r in range(5):
        for d in rounds[r]:
            x0 = x0 + x1
            x1 = _rotl(x1, d)
            x1 = x0 ^ x1
        x0 = x0 + _KS[(r + 1) % 3]
        x1 = x1 + np.uint32((int(_KS[(r + 2) % 3]) + r + 1) & 0xFFFFFFFF)
    return x0 ^ x1


def _gumbel_from_bits(bits, logits):
    float_bits = (bits >> np.uint32(9)) | np.uint32(0x3F800000)
    f = jax.lax.bitcast_convert_type(float_bits, jnp.float32) - np.float32(1.0)
    u = jnp.maximum(f, _TINY)
    g = -jnp.log(-jnp.log(u))
    return g + logits


def _gumbel_plus(logits, x1_init):
    return _gumbel_from_bits(_threefry_bits(x1_init), logits)


_SC_MESH = plsc.VectorSubcoreMesh(core_axis_name="core",
                                  subcore_axis_name="subcore")
_SC_DMA_BLOCK = (8, 512)


def _sc_bits(idx):
    @functools.partial(
        pl.kernel,
        out_type=jax.ShapeDtypeStruct((_ROWS, _V_SC), jnp.uint32),
        mesh=_SC_MESH,
    )
    def sc_kernel(i_hbm, o_hbm):
        def body(i_vmem, o_vmem):
            @pl.loop(0, _SC_DMA_BLOCK[0])
            def _(r):
                @pl.loop(0, _SC_DMA_BLOCK[1], step=64)
                def _(c):
                    for t in range(4):
                        slc = (pl.ds(r, 1), pl.ds(c + t * 16, 16))
                        o_vmem.at[slc][...] = _threefry_bits(
                            i_vmem.at[slc][...])

        pltpu.emit_pipeline(
            body,
            grid=(_ROWS // _SC_DMA_BLOCK[0], _V_SC // _SC_DMA_BLOCK[1]),
            in_specs=[pl.BlockSpec(_SC_DMA_BLOCK, lambda i, j: (i, j))],
            out_specs=[pl.BlockSpec(_SC_DMA_BLOCK, lambda i, j: (i, j))],
            core_axis_name=("core", "subcore"),
            dimension_semantics=(pltpu.PARALLEL, pltpu.PARALLEL),
        )(i_hbm, o_hbm)

    return sc_kernel(idx)


def _head_kernel(x_ref, val_ref, idx_ref, x1_0, acc_val, acc_blk,
                 *, bk, nb, v, rows):
    j = pl.program_id(0)

    @pl.when(j == 0)
    def _():
        cols = jax.lax.broadcasted_iota(jnp.int32, (rows, bk), 1)
        r_iota = jax.lax.broadcasted_iota(jnp.int32, (rows, bk), 0)
        base = (r_iota * v + cols).astype(jnp.uint32) + np.uint32(42)
        x1_0[...] = base
        acc_val[...] = _gumbel_plus(x_ref[...], base)
        acc_blk[...] = jnp.zeros((rows, bk), jnp.int32)

    @pl.when(j > 0)
    def _():
        val = _gumbel_plus(x_ref[...], x1_0[...] + (j * bk).astype(jnp.uint32))
        better = val > acc_val[...]
        acc_blk[...] = jnp.where(better, j, acc_blk[...])
        acc_val[...] = jnp.maximum(val, acc_val[...])

    @pl.when(j == nb - 1)
    def _():
        cols = jax.lax.broadcasted_iota(jnp.int32, (rows, bk), 1)
        m = jnp.max(acc_val[...], axis=1, keepdims=True)
        cand = jnp.where(acc_val[...] == m, acc_blk[...] * bk + cols,
                         jnp.int32(2**31 - 1))
        val_ref[...] = m
        idx_ref[...] = jnp.min(cand, axis=1, keepdims=True)


def _tc_head(logits):
    rows, bk, nb = _ROWS, _BK, _NB_HEAD
    return pl.pallas_call(
        functools.partial(_head_kernel, bk=bk, nb=nb, v=_V, rows=rows),
        grid=(nb,),
        in_specs=[pl.BlockSpec((rows, bk), lambda j: (0, j))],
        out_specs=[pl.BlockSpec((rows, 1), lambda j: (0, 0)),
                   pl.BlockSpec((rows, 1), lambda j: (0, 0))],
        out_shape=[jax.ShapeDtypeStruct((rows, 1), jnp.float32),
                   jax.ShapeDtypeStruct((rows, 1), jnp.int32)],
        scratch_shapes=[
            pltpu.VMEM((rows, bk), jnp.uint32),
            pltpu.VMEM((rows, bk), jnp.float32),
            pltpu.VMEM((rows, bk), jnp.int32),
        ],
    )(logits)


def _tail_kernel(x_ref, b_ref, hval_ref, hidx_ref, o_ref, acc_val, acc_blk,
                 *, bk, nb, v, v0, rows):
    j = pl.program_id(0)

    @pl.when(j == 0)
    def _():
        acc_val[...] = _gumbel_from_bits(b_ref[...], x_ref[...])
        acc_blk[...] = jnp.zeros((rows, bk), jnp.int32)

    @pl.when(jnp.logical_and(j > 0, j < nb - 1))
    def _():
        val = _gumbel_from_bits(b_ref[...], x_ref[...])
        better = val > acc_val[...]
        acc_blk[...] = jnp.where(better, j, acc_blk[...])
        acc_val[...] = jnp.maximum(val, acc_val[...])

    @pl.when(j == nb - 1)
    def _():
        cols = jax.lax.broadcasted_iota(jnp.int32, (rows, bk), 1)
        val = _gumbel_from_bits(b_ref[...], x_ref[...])
        val = jnp.where(v0 + j * bk + cols < v, val, -jnp.inf)
        better = val > acc_val[...]
        a_blk = jnp.where(better, j, acc_blk[...])
        a_val = jnp.maximum(val, acc_val[...])
        m = jnp.max(a_val, axis=1, keepdims=True)
        cand = jnp.where(a_val == m, v0 + a_blk * bk + cols,
                         jnp.int32(2**31 - 1))
        t_idx = jnp.min(cand, axis=1, keepdims=True)
        pick_head = hval_ref[...] >= m
        res = jnp.where(pick_head, hidx_ref[...], t_idx)
        o_ref[...] = res.T


def _tc_tail(logits, bits, head_val, head_idx):
    rows, bk, nb = _ROWS, _BK, _K_SC
    return pl.pallas_call(
        functools.partial(_tail_kernel, bk=bk, nb=nb, v=_V, v0=_V_HEAD,
                          rows=rows),
        grid=(nb,),
        in_specs=[pl.BlockSpec((rows, bk), lambda j: (0, j + _NB_HEAD)),
                  pl.BlockSpec((rows, bk), lambda j: (0, j)),
                  pl.BlockSpec((rows, 1), lambda j: (0, 0)),
                  pl.BlockSpec((rows, 1), lambda j: (0, 0))],
        out_specs=pl.BlockSpec((1, rows), lambda j: (0, 0)),
        out_shape=jax.ShapeDtypeStruct((1, rows), jnp.int32),
        scratch_shapes=[
            pltpu.VMEM((rows, bk), jnp.float32),
            pltpu.VMEM((rows, bk), jnp.int32),
        ],
    )(logits, bits, head_val, head_idx)


def kernel(logits):
    rows, v = logits.shape
    bits = _sc_bits(jnp.asarray(_SC_IDX))
    head_val, head_idx = _tc_head(logits)
    out = _tc_tail(logits, bits, head_val, head_idx)
    return out.reshape(rows)


# speedup vs baseline: 2.6599x; 1.1241x over previous
import functools

import jax
import jax.numpy as jnp
import numpy as np
from jax.experimental import pallas as pl
from jax.experimental.pallas import tpu as pltpu

_BK = 2048

_K1 = np.uint32(0)
_K2 = np.uint32(42)
_K3 = np.uint32(int(_K1) ^ int(_K2) ^ 0x1BD11BDA)
_TINY = np.float32(np.finfo(np.float32).tiny)
_ROT_A = (13, 15, 26, 6)
_ROT_B = (17, 29, 16, 24)
_KS = (_K1, _K2, _K3)


def _rotl(x, d):
    return (x << np.uint32(d)) | (x >> np.uint32(32 - d))


def _threefry_bits(x1):
    x0 = x1
    x1 = _rotl(x1, 13)
    x1 = x0 ^ x1
    rounds = ((15, 26, 6), _ROT_B, _ROT_A, _ROT_B, _ROT_A)
    for r in range(5):
        for d in rounds[r]:
            x0 = x0 + x1
            x1 = _rotl(x1, d)
            x1 = x0 ^ x1
        x0 = x0 + _KS[(r + 1) % 3]
        x1 = x1 + np.uint32((int(_KS[(r + 2) % 3]) + r + 1) & 0xFFFFFFFF)
    return x0 ^ x1


def _gumbel_plus(logits, x1_init):
    bits = _threefry_bits(x1_init)
    float_bits = (bits >> np.uint32(9)) | np.uint32(0x3F800000)
    f = jax.lax.bitcast_convert_type(float_bits, jnp.float32) - np.float32(1.0)
    u = jnp.maximum(f, _TINY)
    g = -jnp.log(-jnp.log(u))
    return g + logits


def _sample_kernel(x_ref, o_ref, x1_0, acc_val, acc_blk, *, bk, nb, v, rows):
    j = pl.program_id(0)

    @pl.when(j == 0)
    def _():
        cols = jax.lax.broadcasted_iota(jnp.int32, (rows, bk), 1)
        r_iota = jax.lax.broadcasted_iota(jnp.int32, (rows, bk), 0)
        base = (r_iota * v + cols).astype(jnp.uint32) + np.uint32(42)
        x1_0[...] = base
        acc_val[...] = _gumbel_plus(x_ref[...], base)
        acc_blk[...] = jnp.zeros((rows, bk), jnp.int32)

    @pl.when(jnp.logical_and(j > 0, j < nb - 1))
    def _():
        val = _gumbel_plus(x_ref[...], x1_0[...] + (j * bk).astype(jnp.uint32))
        better = val > acc_val[...]
        acc_blk[...] = jnp.where(better, j, acc_blk[...])
        acc_val[...] = jnp.maximum(val, acc_val[...])

    @pl.when(j == nb - 1)
    def _():
        cols = jax.lax.broadcasted_iota(jnp.int32, (rows, bk), 1)
        val = _gumbel_plus(x_ref[...], x1_0[...] + (j * bk).astype(jnp.uint32))
        val = jnp.where(cols + j * bk < v, val, -jnp.inf)
        better = val > acc_val[...]
        a_blk = jnp.where(better, j, acc_blk[...])
        a_val = jnp.maximum(val, acc_val[...])
        m = jnp.max(a_val, axis=1, keepdims=True)
        cand = jnp.where(a_val == m, a_blk * bk + cols, jnp.int32(2**31 - 1))
        res = jnp.min(cand, axis=1, keepdims=True)
        o_ref[...] = res.T


def kernel(logits):
    rows, v = logits.shape
    bk = _BK
    nb = pl.cdiv(v, bk)
    out = pl.pallas_call(
        functools.partial(_sample_kernel, bk=bk, nb=nb, v=v, rows=rows),
        grid=(nb,),
        in_specs=[pl.BlockSpec((rows, bk), lambda j: (0, j))],
        out_specs=pl.BlockSpec((1, rows), lambda j: (0, 0)),
        out_shape=jax.ShapeDtypeStruct((1, rows), jnp.int32),
        scratch_shapes=[
            pltpu.VMEM((rows, bk), jnp.uint32),
            pltpu.VMEM((rows, bk), jnp.float32),
            pltpu.VMEM((rows, bk), jnp.int32),
        ],
    )(logits)
    return out.reshape(rows)
